# baseline (speedup 1.0000x reference)
import functools

import jax
import jax.numpy as jnp
from jax import lax
from jax.experimental import pallas as pl
from jax.experimental.pallas import tpu as pltpu
from jax.experimental.pallas import tpu_sc as plsc

BATCH = 16384
HIDDEN = 128
NUM_CORES = 2
NUM_SUBCORES = 16
NW = NUM_CORES * NUM_SUBCORES
B_PER_W = BATCH // NW
CHUNK = 128
N_CHUNKS = B_PER_W // CHUNK
GROUPS = CHUNK // 16


def _dina_body(user_h, item_h, know_h, theta_h, slip_h, guess_h, out_h,
               uidx_v, iidx_v, rows_v, know_v, slip_v, guess_v, sums_v,
               out_v, sem):
    wid = lax.axis_index("s") * NUM_CORES + lax.axis_index("c")
    base = wid * B_PER_W
    iota = lax.iota(jnp.int32, 16)
    for c in range(N_CHUNKS):
        off = base + c * CHUNK
        pltpu.sync_copy(user_h.at[pl.ds(off, CHUNK)], uidx_v)
        pltpu.sync_copy(item_h.at[pl.ds(off, CHUNK)], iidx_v)
        cp_rows = pltpu.async_copy(theta_h.at[uidx_v], rows_v, sem)
        cp_slip = pltpu.async_copy(slip_h.at[iidx_v], slip_v, sem)
        cp_guess = pltpu.async_copy(guess_h.at[iidx_v], guess_v, sem)
        cp_know = pltpu.async_copy(know_h.at[pl.ds(off, CHUNK)], know_v, sem)
        cp_rows.wait()
        cp_slip.wait()
        cp_guess.wait()
        cp_know.wait()

        def group(g, carry):
            gbase = g * 16
            rowi = gbase + iota
            def dstep(d, n_acc):
                ci = jnp.full((16,), d, jnp.int32)
                t = plsc.load_gather(rows_v, [rowi, ci])
                k = plsc.load_gather(know_v, [rowi, ci])
                sig = 1.0 / (1.0 + jnp.exp(-t))
                return n_acc + k * (sig - 0.5)

            n_v = lax.fori_loop(0, HIDDEN, dstep,
                                jnp.zeros((16,), jnp.float32), unroll=8)
            sv = slip_v[pl.ds(gbase, 16)]
            gv = guess_v[pl.ds(gbase, 16)]
            p = 1.0 / (1.0 + jnp.exp(n_v * (-1.0 / 50.0)))
            sl = 0.4 / (1.0 + jnp.exp(-sv))
            gs = 0.4 / (1.0 + jnp.exp(-gv))
            res = (1.0 - sl) * p + gs * (1.0 - p)
            out_v[pl.ds(gbase, 16)] = res
            return carry

        lax.fori_loop(0, GROUPS, group, 0)
        pltpu.sync_copy(out_v, out_h.at[pl.ds(off, CHUNK)])


def kernel(user, item, knowledge, theta_w, slip_w, guess_w):
    mesh = plsc.VectorSubcoreMesh(core_axis_name="c", subcore_axis_name="s")
    slip_flat = slip_w.reshape(-1)
    guess_flat = guess_w.reshape(-1)
    run = functools.partial(
        pl.kernel,
        mesh=mesh,
        compiler_params=pltpu.CompilerParams(
            needs_layout_passes=False, use_tc_tiling_on_sc=False),
        out_type=jax.ShapeDtypeStruct((BATCH,), jnp.float32),
        scratch_types=[
            pltpu.VMEM((CHUNK,), jnp.int32),
            pltpu.VMEM((CHUNK,), jnp.int32),
            pltpu.VMEM((CHUNK, HIDDEN), jnp.float32),
            pltpu.VMEM((CHUNK, HIDDEN), jnp.float32),
            pltpu.VMEM((CHUNK,), jnp.float32),
            pltpu.VMEM((CHUNK,), jnp.float32),
            pltpu.VMEM((256,), jnp.float32),
            pltpu.VMEM((CHUNK,), jnp.float32),
            pltpu.SemaphoreType.DMA,
        ],
    )(_dina_body)
    return run(user, item, knowledge, theta_w, slip_flat, guess_flat)
